# baseline (speedup 1.0000x reference)
import functools
import jax
import jax.numpy as jnp
from jax import lax
from jax.experimental import pallas as pl
from jax.experimental.pallas import tpu as pltpu
from jax.experimental.pallas import tpu_sc as plsc

_SCALE = 1.0 + 0.1
_NC = 2
_NS = 16
_NW = _NC * _NS
_N = 128
_F = 256
_LANES = 16


def _sc_body(x_hbm, sz_hbm, o_hbm, sz_v, buf, sem):
    wid = lax.axis_index("s") * _NC + lax.axis_index("c")
    tiles_per_worker = 8
    base_t = wid * tiles_per_worker
    pltpu.sync_copy(sz_hbm.at[pl.ds(base_t, _LANES)], sz_v)
    szv = sz_v[...]
    for j in range(tiles_per_worker):
        s = szv[j]
        row0 = (base_t + j) * _N
        pltpu.async_copy(x_hbm.at[pl.ds(row0, _N)], buf, sem).wait()

        @pl.loop(0, _N)
        def _(r):
            sc = jnp.where(r < s, _SCALE, 0.0).astype(jnp.float32)
            for c in range(_F // _LANES):
                sl = pl.ds(c * _LANES, _LANES)
                buf[r, sl] = buf[r, sl] * sc

        pltpu.async_copy(buf, o_hbm.at[pl.ds(row0, _N)], sem).wait()


def kernel(input_tensor, batch_set_size):
    B, T, N, F = input_tensor.shape
    BT = B * T
    x = input_tensor.reshape(BT * N, F)
    sizes = jnp.pad(batch_set_size.reshape(BT), (0, _LANES))

    mesh = plsc.VectorSubcoreMesh(core_axis_name="c", subcore_axis_name="s")
    run = functools.partial(
        pl.kernel,
        mesh=mesh,
        out_type=jax.ShapeDtypeStruct((BT * N, F), input_tensor.dtype),
        scratch_types=[
            pltpu.VMEM((_LANES,), jnp.int32),
            pltpu.VMEM((_N, _F), jnp.float32),
            pltpu.SemaphoreType.DMA,
        ],
    )(_sc_body)
    out = run(x, sizes)
    return out.reshape(B, T, N, F)
